# Initial kernel scaffold; baseline (speedup 1.0000x reference)
import jax
import jax.numpy as jnp
from jax.experimental import pallas as pl


def kernel(x, edge_index, batch, W1, b1, W2, b2, Wout, bout):
    raise NotImplementedError("write your pallas kernel here")


# speedup vs baseline: 134.8434x; 134.8434x over previous
import functools

import jax
import jax.numpy as jnp
import numpy as np
from jax import lax
from jax.experimental import pallas as pl
from jax.experimental.pallas import tpu as pltpu
from jax.experimental.pallas import tpu_sc as plsc

F32 = jnp.float32
I32 = jnp.int32

NC = 2
NS = 16
NW = NC * NS
K = 128
G = 64


def _deg_body(dst_hbm, out_hbm, idx_v, hist, *, nchunks):
    c = lax.axis_index("c")
    s = lax.axis_index("s")
    w = c * jnp.int32(NS) + s
    npad = hist.shape[0]

    def zero(r, carry):
        hist[pl.ds(r * jnp.int32(16), 16)] = jnp.zeros((16,), F32)
        return carry

    lax.fori_loop(jnp.int32(0), jnp.int32(npad // 16), zero, jnp.int32(0))
    base = w * jnp.int32(nchunks * K)

    def body(i, carry):
        off = pl.multiple_of(base + i * jnp.int32(K), K)
        pltpu.sync_copy(dst_hbm.at[pl.ds(off, K)], idx_v)
        for k in range(K // 16):
            v = idx_v[pl.ds(jnp.int32(k * 16), 16)]
            plsc.addupdate_scatter(hist, [v], jnp.ones((16,), F32))
        return carry

    lax.fori_loop(jnp.int32(0), jnp.int32(nchunks), body, jnp.int32(0))
    pltpu.sync_copy(hist, out_hbm.at[c, s])


def _agg_body(src_hbm, dst_hbm, y_hbm, zeros_hbm, out_hbm,
              sidx, didx, rows, acc, sem, *, nchunks):
    c = lax.axis_index("c")
    s = lax.axis_index("s")
    w = c * jnp.int32(NS) + s
    rows_per_sub = acc.shape[0] // NS
    pltpu.sync_copy(zeros_hbm, acc.at[pl.ds(s * jnp.int32(rows_per_sub), rows_per_sub)])
    plsc.subcore_barrier()
    base = w * jnp.int32(nchunks * K)

    def body(i, carry):
        off = pl.multiple_of(base + i * jnp.int32(K), K)
        pltpu.sync_copy(src_hbm.at[pl.ds(off, K)], sidx)
        pltpu.sync_copy(dst_hbm.at[pl.ds(off, K)], didx)
        pltpu.async_copy(y_hbm.at[sidx], rows, sem).wait()
        pltpu.sync_copy(rows, acc.at[didx], add=True)
        return carry

    lax.fori_loop(jnp.int32(0), jnp.int32(nchunks), body, jnp.int32(0))
    plsc.subcore_barrier()
    pltpu.sync_copy(acc.at[pl.ds(s * jnp.int32(rows_per_sub), rows_per_sub)],
                    out_hbm.at[c, pl.ds(s * jnp.int32(rows_per_sub), rows_per_sub)])


def _dinv_from_parts(degp_ref):
    deg = jnp.sum(degp_ref[...], axis=0) + 1.0
    return lax.rsqrt(deg)


def _mm_scale_kernel(x_ref, w_ref, degp_ref, o_ref):
    dinv = _dinv_from_parts(degp_ref)
    o_ref[...] = jnp.dot(x_ref[...], w_ref[...], preferred_element_type=F32, precision=lax.Precision.HIGHEST) * dinv


def _combine_kernel(aggp_ref, y_ref, degp_ref, b_ref, w_ref, o_ref):
    dinv = _dinv_from_parts(degp_ref)
    t = (aggp_ref[0] + aggp_ref[1] + y_ref[...]) * dinv + b_ref[...]
    h = jnp.maximum(t, 0.0)
    o_ref[...] = jnp.dot(h, w_ref[...], preferred_element_type=F32, precision=lax.Precision.HIGHEST) * dinv


def _final_kernel(aggp_ref, y_ref, degp_ref, b_ref, batch_ref, wout_ref,
                  bout_ref, o_ref, sums, cnts):
    i = pl.program_id(0)

    @pl.when(i == 0)
    def _init():
        sums[...] = jnp.zeros_like(sums)
        cnts[...] = jnp.zeros_like(cnts)

    dinv = _dinv_from_parts(degp_ref)
    t = (aggp_ref[0] + aggp_ref[1] + y_ref[...]) * dinv + b_ref[...]
    h = jnp.maximum(t, 0.0)
    oh = (batch_ref[...] == lax.broadcasted_iota(I32, (1, G), 1)).astype(F32)
    dn = (((0,), (0,)), ((), ()))
    sums[...] += lax.dot_general(oh, h, dn, preferred_element_type=F32, precision=lax.Precision.HIGHEST)
    cnts[...] += lax.dot_general(oh, jnp.ones_like(h), dn, preferred_element_type=F32, precision=lax.Precision.HIGHEST)

    @pl.when(i == pl.num_programs(0) - 1)
    def _fin():
        mean = sums[...] / jnp.maximum(cnts[...], 1.0)
        o_ref[...] = jnp.dot(mean, wout_ref[...], preferred_element_type=F32, precision=lax.Precision.HIGHEST) + bout_ref[...]


def _sc_mesh():
    return plsc.VectorSubcoreMesh(core_axis_name="c", subcore_axis_name="s",
                                  num_cores=NC, num_subcores=NS)


def _deg_parts(dst_p, npad, nch):
    call = pl.kernel(
        functools.partial(_deg_body, nchunks=nch),
        out_type=jax.ShapeDtypeStruct((NC, NS, npad), F32),
        mesh=_sc_mesh(),
        scratch_types=[
            pltpu.VMEM((K,), I32),
            pltpu.VMEM((npad,), F32),
        ],
        compiler_params=pltpu.CompilerParams(needs_layout_passes=False),
    )
    return call(dst_p)


def _agg_parts(src_p, dst_p, y, zeros_wide, npad, nch):
    h = y.shape[1]
    call = pl.kernel(
        functools.partial(_agg_body, nchunks=nch),
        out_type=jax.ShapeDtypeStruct((NC, npad, h), F32),
        mesh=_sc_mesh(),
        scratch_types=[
            pltpu.VMEM((K,), I32),
            pltpu.VMEM((K,), I32),
            pltpu.VMEM((K, h), F32),
            pltpu.VMEM_SHARED((npad, h), F32),
            pltpu.SemaphoreType.DMA,
        ],
    )
    return call(src_p, dst_p, y, zeros_wide)


_Z = np.int32(0)


def _im_i0(i):
    return i, _Z


def _im_0i0(i):
    return _Z, i, _Z


def _im_00(i):
    return _Z, _Z


def kernel(x, edge_index, batch, W1, b1, W2, b2, Wout, bout):
    N, D = x.shape
    H = W1.shape[1]
    E = edge_index.shape[1]
    NPAD = (N // 1024 + 1) * 1024
    R = 1024
    grid = NPAD // R
    epc = NW * K
    nch = -(-E // epc)
    e_pad = nch * epc

    x32 = x.astype(F32)
    src_p = jnp.concatenate([edge_index[0].astype(I32),
                             jnp.zeros((e_pad - E,), I32)])
    dst_p = jnp.concatenate([edge_index[1].astype(I32),
                             jnp.full((e_pad - E,), N, I32)])
    x_p = jnp.concatenate([x32, jnp.zeros((NPAD - N, D), F32)])
    batch_p = jnp.concatenate([batch.astype(I32),
                               jnp.full((NPAD - N,), G, I32)]).reshape(NPAD, 1)
    zeros_wide = jnp.zeros((NPAD // NS, H), F32)
    W1f = W1.astype(F32)
    W2f = W2.astype(F32)
    Woutf = Wout.astype(F32)
    b1r = b1.astype(F32).reshape(1, H)
    b2r = b2.astype(F32).reshape(1, H)
    boutr = bout.astype(F32).reshape(1, 1)

    degparts = _deg_parts(dst_p, NPAD, nch).reshape(NW, NPAD, 1)

    y1 = pl.pallas_call(
        _mm_scale_kernel,
        grid=(grid,),
        in_specs=[
            pl.BlockSpec((R, D), _im_i0),
            pl.BlockSpec((D, H), _im_00),
            pl.BlockSpec((NW, R, 1), _im_0i0),
        ],
        out_specs=pl.BlockSpec((R, H), _im_i0),
        out_shape=jax.ShapeDtypeStruct((NPAD, H), F32),
    )(x_p, W1f, degparts)

    agg1 = _agg_parts(src_p, dst_p, y1, zeros_wide, NPAD, nch)

    y2 = pl.pallas_call(
        _combine_kernel,
        grid=(grid,),
        in_specs=[
            pl.BlockSpec((NC, R, H), _im_0i0),
            pl.BlockSpec((R, H), _im_i0),
            pl.BlockSpec((NW, R, 1), _im_0i0),
            pl.BlockSpec((1, H), _im_00),
            pl.BlockSpec((H, H), _im_00),
        ],
        out_specs=pl.BlockSpec((R, H), _im_i0),
        out_shape=jax.ShapeDtypeStruct((NPAD, H), F32),
    )(agg1, y1, degparts, b1r, W2f)

    agg2 = _agg_parts(src_p, dst_p, y2, zeros_wide, NPAD, nch)

    out = pl.pallas_call(
        _final_kernel,
        grid=(grid,),
        in_specs=[
            pl.BlockSpec((NC, R, H), _im_0i0),
            pl.BlockSpec((R, H), _im_i0),
            pl.BlockSpec((NW, R, 1), _im_0i0),
            pl.BlockSpec((1, H), _im_00),
            pl.BlockSpec((R, 1), _im_i0),
            pl.BlockSpec((H, 1), _im_00),
            pl.BlockSpec((1, 1), _im_00),
        ],
        out_specs=pl.BlockSpec((G, 1), _im_00),
        out_shape=jax.ShapeDtypeStruct((G, 1), F32),
        scratch_shapes=[
            pltpu.VMEM((G, H), F32),
            pltpu.VMEM((G, H), F32),
        ],
    )(agg2, y2, degparts, b2r, batch_p, Woutf, boutr)

    return out.astype(jnp.float64)
